# baseline (speedup 1.0000x reference)
import jax
import jax.numpy as jnp
from jax.experimental import pallas as pl
from jax.experimental.pallas import tpu as pltpu

_N = 10000
_F_IN = 128
_F_OUT = 64
_BLOCK = 1000


def _gru_kernel(x_ref, h_ref, wz_ref, bz_ref, wr_ref, br_ref, wh_ref, bh_ref,
                lw_ref, lb_ref, out_ref, hnew_ref):
    x = x_ref[...]
    h = h_ref[...]
    wz = wz_ref[0] + wz_ref[1]
    wr = wr_ref[0] + wr_ref[1]
    wh = wh_ref[0] + wh_ref[1]

    def mm(a, b):
        return jax.lax.dot_general(a, b, (((1,), (0,)), ((), ())),
                                   preferred_element_type=jnp.float32)

    z = jax.nn.sigmoid(mm(x, wz[:_F_IN]) + mm(h, wz[_F_IN:]) + bz_ref[...])
    r = jax.nn.sigmoid(mm(x, wr[:_F_IN]) + mm(h, wr[_F_IN:]) + br_ref[...])
    ht = jnp.tanh(mm(x, wh[:_F_IN]) + mm(h * r, wh[_F_IN:]) + bh_ref[...])
    h_new = z * h + (1.0 - z) * ht
    hnew_ref[...] = h_new
    relu_h = jnp.maximum(h_new, 0.0)
    out_ref[...] = jnp.sum(relu_h * lw_ref[...], axis=1, keepdims=True) + lb_ref[...]


def kernel(x, edge_index, edge_weight, memory, W_z, b_z, W_r, b_r, W_h, b_h,
           lin_w, lin_b):
    del edge_index, edge_weight
    wz = W_z.reshape(2, _F_IN + _F_OUT, _F_OUT)
    wr = W_r.reshape(2, _F_IN + _F_OUT, _F_OUT)
    wh = W_h.reshape(2, _F_IN + _F_OUT, _F_OUT)
    bz = b_z.reshape(1, _F_OUT)
    br = b_r.reshape(1, _F_OUT)
    bh = b_h.reshape(1, _F_OUT)
    lw = lin_w.reshape(1, _F_OUT)
    lb = lin_b.reshape(1, 1)

    grid = _N // _BLOCK
    row_spec = lambda w: pl.BlockSpec((_BLOCK, w), lambda i: (i, 0))
    full = lambda shape: pl.BlockSpec(shape, lambda i: (0,) * len(shape))

    out, h_new = pl.pallas_call(
        _gru_kernel,
        grid=(grid,),
        in_specs=[
            row_spec(_F_IN),
            row_spec(_F_OUT),
            full((2, _F_IN + _F_OUT, _F_OUT)),
            full((1, _F_OUT)),
            full((2, _F_IN + _F_OUT, _F_OUT)),
            full((1, _F_OUT)),
            full((2, _F_IN + _F_OUT, _F_OUT)),
            full((1, _F_OUT)),
            full((1, _F_OUT)),
            full((1, 1)),
        ],
        out_specs=[
            pl.BlockSpec((_BLOCK, 1), lambda i: (i, 0)),
            row_spec(_F_OUT),
        ],
        out_shape=[
            jax.ShapeDtypeStruct((_N, 1), jnp.float32),
            jax.ShapeDtypeStruct((_N, _F_OUT), jnp.float32),
        ],
        compiler_params=pltpu.CompilerParams(
            dimension_semantics=("arbitrary",),
        ),
    )(x, memory, wz, bz, wr, br, wh, bh, lw, lb)
    return (out, h_new)


# speedup vs baseline: 1.0612x; 1.0612x over previous
import jax
import jax.numpy as jnp
from jax.experimental import pallas as pl
from jax.experimental.pallas import tpu as pltpu

_N = 10000
_F_IN = 128
_F_OUT = 64
_BLOCK = 5000


def _gru_kernel(x_ref, h_ref, wz_ref, bz_ref, wr_ref, br_ref, wh_ref, bh_ref,
                lw_ref, lb_ref, out_ref, hnew_ref):
    x = x_ref[...]
    h = h_ref[...]
    wz = wz_ref[0] + wz_ref[1]
    wr = wr_ref[0] + wr_ref[1]
    wh = wh_ref[0] + wh_ref[1]

    def mm(a, b):
        return jax.lax.dot_general(a, b, (((1,), (0,)), ((), ())),
                                   preferred_element_type=jnp.float32)

    z = jax.nn.sigmoid(mm(x, wz[:_F_IN]) + mm(h, wz[_F_IN:]) + bz_ref[...])
    r = jax.nn.sigmoid(mm(x, wr[:_F_IN]) + mm(h, wr[_F_IN:]) + br_ref[...])
    ht = jnp.tanh(mm(x, wh[:_F_IN]) + mm(h * r, wh[_F_IN:]) + bh_ref[...])
    h_new = z * h + (1.0 - z) * ht
    hnew_ref[...] = h_new
    relu_h = jnp.maximum(h_new, 0.0)
    out_ref[...] = jnp.sum(relu_h * lw_ref[...], axis=1, keepdims=True) + lb_ref[...]


def kernel(x, edge_index, edge_weight, memory, W_z, b_z, W_r, b_r, W_h, b_h,
           lin_w, lin_b):
    del edge_index, edge_weight
    wz = W_z.reshape(2, _F_IN + _F_OUT, _F_OUT)
    wr = W_r.reshape(2, _F_IN + _F_OUT, _F_OUT)
    wh = W_h.reshape(2, _F_IN + _F_OUT, _F_OUT)
    bz = b_z.reshape(1, _F_OUT)
    br = b_r.reshape(1, _F_OUT)
    bh = b_h.reshape(1, _F_OUT)
    lw = lin_w.reshape(1, _F_OUT)
    lb = lin_b.reshape(1, 1)

    grid = _N // _BLOCK
    row_spec = lambda w: pl.BlockSpec((_BLOCK, w), lambda i: (i, 0))
    full = lambda shape: pl.BlockSpec(shape, lambda i: (0,) * len(shape))

    out, h_new = pl.pallas_call(
        _gru_kernel,
        grid=(grid,),
        in_specs=[
            row_spec(_F_IN),
            row_spec(_F_OUT),
            full((2, _F_IN + _F_OUT, _F_OUT)),
            full((1, _F_OUT)),
            full((2, _F_IN + _F_OUT, _F_OUT)),
            full((1, _F_OUT)),
            full((2, _F_IN + _F_OUT, _F_OUT)),
            full((1, _F_OUT)),
            full((1, _F_OUT)),
            full((1, 1)),
        ],
        out_specs=[
            pl.BlockSpec((_BLOCK, 1), lambda i: (i, 0)),
            row_spec(_F_OUT),
        ],
        out_shape=[
            jax.ShapeDtypeStruct((_N, 1), jnp.float32),
            jax.ShapeDtypeStruct((_N, _F_OUT), jnp.float32),
        ],
        compiler_params=pltpu.CompilerParams(
            dimension_semantics=("arbitrary",),
        ),
    )(x, memory, wz, bz, wr, br, wh, bh, lw, lb)
    return (out, h_new)


# speedup vs baseline: 1.1075x; 1.0436x over previous
import jax
import jax.numpy as jnp
from jax.experimental import pallas as pl
from jax.experimental.pallas import tpu as pltpu

_N = 10000
_F_IN = 128
_F_OUT = 64
_BLOCK = 2000


def _gru_kernel(x_ref, h_ref, wz_ref, bz_ref, wr_ref, br_ref, wh_ref, bh_ref,
                lw_ref, lb_ref, out_ref, hnew_ref):
    x = x_ref[...]
    h = h_ref[...]
    wz = wz_ref[0] + wz_ref[1]
    wr = wr_ref[0] + wr_ref[1]
    wh = wh_ref[0] + wh_ref[1]

    def mm(a, b):
        return jax.lax.dot_general(a, b, (((1,), (0,)), ((), ())),
                                   preferred_element_type=jnp.float32)

    z = jax.nn.sigmoid(mm(x, wz[:_F_IN]) + mm(h, wz[_F_IN:]) + bz_ref[...])
    r = jax.nn.sigmoid(mm(x, wr[:_F_IN]) + mm(h, wr[_F_IN:]) + br_ref[...])
    ht = jnp.tanh(mm(x, wh[:_F_IN]) + mm(h * r, wh[_F_IN:]) + bh_ref[...])
    h_new = z * h + (1.0 - z) * ht
    hnew_ref[...] = h_new
    relu_h = jnp.maximum(h_new, 0.0)
    out_ref[...] = jnp.sum(relu_h * lw_ref[...], axis=1, keepdims=True) + lb_ref[...]


def kernel(x, edge_index, edge_weight, memory, W_z, b_z, W_r, b_r, W_h, b_h,
           lin_w, lin_b):
    del edge_index, edge_weight
    wz = W_z.reshape(2, _F_IN + _F_OUT, _F_OUT)
    wr = W_r.reshape(2, _F_IN + _F_OUT, _F_OUT)
    wh = W_h.reshape(2, _F_IN + _F_OUT, _F_OUT)
    bz = b_z.reshape(1, _F_OUT)
    br = b_r.reshape(1, _F_OUT)
    bh = b_h.reshape(1, _F_OUT)
    lw = lin_w.reshape(1, _F_OUT)
    lb = lin_b.reshape(1, 1)

    grid = _N // _BLOCK
    row_spec = lambda w: pl.BlockSpec((_BLOCK, w), lambda i: (i, 0))
    full = lambda shape: pl.BlockSpec(shape, lambda i: (0,) * len(shape))

    out, h_new = pl.pallas_call(
        _gru_kernel,
        grid=(grid,),
        in_specs=[
            row_spec(_F_IN),
            row_spec(_F_OUT),
            full((2, _F_IN + _F_OUT, _F_OUT)),
            full((1, _F_OUT)),
            full((2, _F_IN + _F_OUT, _F_OUT)),
            full((1, _F_OUT)),
            full((2, _F_IN + _F_OUT, _F_OUT)),
            full((1, _F_OUT)),
            full((1, _F_OUT)),
            full((1, 1)),
        ],
        out_specs=[
            pl.BlockSpec((_BLOCK, 1), lambda i: (i, 0)),
            row_spec(_F_OUT),
        ],
        out_shape=[
            jax.ShapeDtypeStruct((_N, 1), jnp.float32),
            jax.ShapeDtypeStruct((_N, _F_OUT), jnp.float32),
        ],
        compiler_params=pltpu.CompilerParams(
            dimension_semantics=("parallel",),
        ),
    )(x, memory, wz, bz, wr, br, wh, bh, lw, lb)
    return (out, h_new)
